# baseline (speedup 1.0000x reference)
import functools

import jax
import jax.numpy as jnp
from jax import lax
from jax.experimental import pallas as pl
from jax.experimental.pallas import tpu as pltpu
from jax.experimental.pallas import tpu_sc as plsc

N_NODES = 10000
NP = 10240
N_CORES = 2
N_SUBCORES = 16
NW = N_CORES * N_SUBCORES
CHUNK = 128
CPT = 80
GRP = 16
EP = NW * CPT * CHUNK
ROWS_PER_TILE = NP // N_SUBCORES


def _mm_body(h_ref, wl_ref, wr_ref, b_ref, y_ref, r_ref):
    h = h_ref[...]
    y_ref[...] = jnp.dot(h, wl_ref[...], preferred_element_type=jnp.float32)
    r_ref[...] = (jnp.dot(h, wr_ref[...], preferred_element_type=jnp.float32)
                  + b_ref[0:1, :])


def _matmuls(h, W_l, W_r, b):
    ci, co = W_l.shape
    blk = 512
    b8 = jnp.broadcast_to(b.reshape(1, co), (8, co))
    return pl.pallas_call(
        _mm_body,
        grid=(NP // blk,),
        in_specs=[
            pl.BlockSpec((blk, ci), lambda i: (i, 0)),
            pl.BlockSpec((ci, co), lambda i: (0, 0)),
            pl.BlockSpec((ci, co), lambda i: (0, 0)),
            pl.BlockSpec((8, co), lambda i: (0, 0)),
        ],
        out_specs=[
            pl.BlockSpec((blk, co), lambda i: (i, 0)),
            pl.BlockSpec((blk, co), lambda i: (i, 0)),
        ],
        out_shape=[
            jax.ShapeDtypeStruct((NP, co), jnp.float32),
            jax.ShapeDtypeStruct((NP, co), jnp.float32),
        ],
    )(h, W_l, W_r, b8)


def _cmm_body(p_ref, c_ref, r_ref, wl_ref, wr_ref, b_ref, y_ref, r2_ref):
    p = p_ref[0] + p_ref[1]
    cnt = c_ref[0, :, 0:1] + c_ref[1, :, 0:1]
    inv = 1.0 / jnp.maximum(cnt, 1.0)
    h = jnp.maximum(p * inv + r_ref[...], 0.0)
    y_ref[...] = jnp.dot(h, wl_ref[...], preferred_element_type=jnp.float32)
    r2_ref[...] = (jnp.dot(h, wr_ref[...], preferred_element_type=jnp.float32)
                   + b_ref[0:1, :])


def _combine_mm(P, C, R, W_l, W_r, b):
    ci = P.shape[2]
    co = W_l.shape[1]
    blk = 512
    b8 = jnp.broadcast_to(b.reshape(1, co), (8, co))
    return pl.pallas_call(
        _cmm_body,
        grid=(NP // blk,),
        in_specs=[
            pl.BlockSpec((2, blk, ci), lambda i: (0, i, 0)),
            pl.BlockSpec((2, blk, 16), lambda i: (0, i, 0)),
            pl.BlockSpec((blk, ci), lambda i: (i, 0)),
            pl.BlockSpec((ci, co), lambda i: (0, 0)),
            pl.BlockSpec((ci, co), lambda i: (0, 0)),
            pl.BlockSpec((8, co), lambda i: (0, 0)),
        ],
        out_specs=[
            pl.BlockSpec((blk, co), lambda i: (i, 0)),
            pl.BlockSpec((blk, co), lambda i: (i, 0)),
        ],
        out_shape=[
            jax.ShapeDtypeStruct((NP, co), jnp.float32),
            jax.ShapeDtypeStruct((NP, co), jnp.float32),
        ],
    )(P, C, R, W_l, W_r, b8)


def _combine_body(p_ref, c_ref, r_ref, o_ref, *, act):
    p = p_ref[0] + p_ref[1]
    cnt = c_ref[0, :, 0:1] + c_ref[1, :, 0:1]
    inv = 1.0 / jnp.maximum(cnt, 1.0)
    z = p * inv + r_ref[...]
    if act == "relu":
        o_ref[...] = jnp.maximum(z, 0.0)
    else:
        col = lax.broadcasted_iota(jnp.int32, z.shape, 1)
        valid = col < 40
        zm = jnp.where(valid, z, -jnp.inf)
        m = jnp.max(zm, axis=1, keepdims=True)
        e = jnp.where(valid, jnp.exp(z - m), 0.0)
        lse = jnp.log(jnp.sum(e, axis=1, keepdims=True)) + m
        o_ref[...] = z - lse


def _combine(P, C, R, act):
    co = P.shape[2]
    blk = 512
    return pl.pallas_call(
        functools.partial(_combine_body, act=act),
        grid=(NP // blk,),
        in_specs=[
            pl.BlockSpec((2, blk, co), lambda i: (0, i, 0)),
            pl.BlockSpec((2, blk, 16), lambda i: (0, i, 0)),
            pl.BlockSpec((blk, co), lambda i: (i, 0)),
        ],
        out_specs=pl.BlockSpec((blk, co), lambda i: (i, 0)),
        out_shape=jax.ShapeDtypeStruct((NP, co), jnp.float32),
    )(P, C, R)


def _seg_sum(y, src_g, dst_g):
    co = y.shape[1]
    mesh = plsc.VectorSubcoreMesh(
        core_axis_name="c", subcore_axis_name="s",
        num_cores=N_CORES, num_subcores=N_SUBCORES)

    out_type = [jax.ShapeDtypeStruct((N_CORES, NP, co), jnp.float32)]
    scratch = [
        pltpu.VMEM((GRP, CHUNK), jnp.int32),
        pltpu.VMEM((GRP, CHUNK), jnp.int32),
        pltpu.VMEM((GRP, CHUNK), jnp.int32),
        pltpu.VMEM((GRP, CHUNK), jnp.int32),
        pltpu.VMEM((GRP, CHUNK), jnp.int32),
        pltpu.VMEM((GRP, CHUNK), jnp.int32),
        pltpu.VMEM((CHUNK, co), jnp.float32),
        pltpu.VMEM((CHUNK, co), jnp.float32),
        pltpu.VMEM_SHARED((NP, co), jnp.float32),
        pltpu.SemaphoreType.DMA,
        pltpu.SemaphoreType.DMA,
        pltpu.SemaphoreType.DMA,
        pltpu.SemaphoreType.DMA,
        pltpu.SemaphoreType.DMA,
        pltpu.SemaphoreType.DMA,
        pltpu.SemaphoreType.DMA,
    ]

    zeros = jnp.zeros((CHUNK, co), jnp.float32)
    ins = [y, src_g, dst_g, zeros]

    @functools.partial(
        pl.kernel, out_type=out_type, mesh=mesh, scratch_types=scratch,
        compiler_params=pltpu.CompilerParams(use_tc_tiling_on_sc=(co == 128)))
    def body(y_hbm, src_hbm, dst_hbm, z_hbm, out_hbm,
             src0, src1, src2, dst0, dst1, dst2, rows_a, rows_b, acc,
             gsa, gsb, ssa, ssb, is0, is1, is2):
        c = lax.axis_index("c")
        s = lax.axis_index("s")
        wid = c * N_SUBCORES + s
        r0 = s * ROWS_PER_TILE
        n_blk = ROWS_PER_TILE // CHUNK
        n_grp = CPT // GRP
        rows = (rows_a, rows_b)
        gsem = (gsa, gsb)
        ssem = (ssa, ssb)
        srcb = (src0, src1, src2)
        dstb = (dst0, dst1, dst2)
        isem = (is0, is1, is2)

        def fire_idx(g):
            b = g % 3
            d1 = pltpu.async_copy(src_hbm.at[wid, pl.ds(g * GRP, GRP)],
                                  srcb[b], isem[b])
            d2 = pltpu.async_copy(dst_hbm.at[wid, pl.ds(g * GRP, GRP)],
                                  dstb[b], isem[b])
            return (d1, d2)

        def wait_idx(ds):
            ds[0].wait()
            ds[1].wait()

        ids = [None] * n_grp
        ids[0] = fire_idx(0)
        ids[1] = fire_idx(1)
        pltpu.sync_copy(z_hbm, rows_a)
        zds = [pltpu.async_copy(rows_a, acc.at[pl.ds(r0 + j * CHUNK, CHUNK)],
                                ssa)
               for j in range(n_blk)]
        for d in zds:
            d.wait()
        wait_idx(ids[0])
        gd = [None] * CPT
        gd[0] = pltpu.async_copy(y_hbm.at[srcb[0].at[0]], rows[0], gsem[0])
        plsc.subcore_barrier()

        sd = [None] * CPT
        for j in range(CPT):
            p = j % 2
            g = j // GRP
            jj = j % GRP
            if j + 1 < CPT:
                gn = (j + 1) // GRP
                jn = (j + 1) % GRP
                if jn == 0:
                    wait_idx(ids[gn])
                if j >= 1:
                    sd[j - 1].wait()
                gd[j + 1] = pltpu.async_copy(
                    y_hbm.at[srcb[gn % 3].at[jn]], rows[1 - p], gsem[1 - p])
                if jn == 0 and gn + 1 < n_grp:
                    ids[gn + 1] = fire_idx(gn + 1)
            gd[j].wait()
            sd[j] = pltpu.async_copy(
                rows[p], acc.at[dstb[g % 3].at[jj]], ssem[p], add=True)
        sd[CPT - 2].wait()
        sd[CPT - 1].wait()
        plsc.subcore_barrier()

        ids = [None] * n_blk
        ods = [None] * n_blk
        ids[0] = pltpu.async_copy(acc.at[pl.ds(r0, CHUNK)], rows_a, gsa)
        for j in range(n_blk):
            p = j % 2
            q = 1 - p
            if j + 1 < n_blk:
                if j >= 1:
                    ods[j - 1].wait()
                ids[j + 1] = pltpu.async_copy(
                    acc.at[pl.ds(r0 + (j + 1) * CHUNK, CHUNK)],
                    rows[q], gsem[q])
            ids[j].wait()
            ods[j] = pltpu.async_copy(
                rows[p], out_hbm.at[c, pl.ds(r0 + j * CHUNK, CHUNK)],
                ssem[p])
        ods[n_blk - 2].wait()
        ods[n_blk - 1].wait()

    return body(*ins)


def _seg_cnt(dst_g):
    mesh = plsc.VectorSubcoreMesh(
        core_axis_name="c", subcore_axis_name="s",
        num_cores=N_CORES, num_subcores=N_SUBCORES)

    out_type = [jax.ShapeDtypeStruct((N_CORES, NP, 16), jnp.float32)]
    scratch = [
        pltpu.VMEM((GRP, CHUNK), jnp.int32),
        pltpu.VMEM((GRP, CHUNK), jnp.int32),
        pltpu.VMEM((CHUNK, 16), jnp.float32),
        pltpu.VMEM((CHUNK, 16), jnp.float32),
        pltpu.VMEM_SHARED((NP, 16), jnp.float32),
        pltpu.SemaphoreType.DMA,
        pltpu.SemaphoreType.DMA,
        pltpu.SemaphoreType.DMA,
        pltpu.SemaphoreType.DMA,
    ]

    @functools.partial(
        pl.kernel, out_type=out_type, mesh=mesh, scratch_types=scratch,
        compiler_params=pltpu.CompilerParams(use_tc_tiling_on_sc=False))
    def body(dst_hbm, z_hbm, ones_hbm, cnt_hbm,
             dst_a, dst_b, ones_v, st_v, acc_c, isa, isb, ssa, ssb):
        c = lax.axis_index("c")
        s = lax.axis_index("s")
        wid = c * N_SUBCORES + s
        r0 = s * ROWS_PER_TILE
        n_blk = ROWS_PER_TILE // CHUNK
        n_grp = CPT // GRP
        dstb = (dst_a, dst_b)
        isem = (isa, isb)
        ssem = (ssa, ssb)
        pltpu.sync_copy(z_hbm, st_v)
        pltpu.sync_copy(ones_hbm, ones_v)
        zds = [pltpu.async_copy(st_v, acc_c.at[pl.ds(r0 + j * CHUNK, CHUNK)],
                                ssa)
               for j in range(n_blk)]
        for d in zds:
            d.wait()
        plsc.subcore_barrier()

        ids = [None] * n_grp
        sds = [[None] * GRP for _ in range(n_grp)]
        ids[0] = pltpu.async_copy(dst_hbm.at[wid, pl.ds(0, GRP)], dst_a,
                                  isem[0])
        for g in range(n_grp):
            p = g % 2
            if g + 1 < n_grp:
                if g >= 1:
                    for d in sds[g - 1]:
                        d.wait()
                ids[g + 1] = pltpu.async_copy(
                    dst_hbm.at[wid, pl.ds((g + 1) * GRP, GRP)],
                    dstb[1 - p], isem[1 - p])
            ids[g].wait()
            for j in range(GRP):
                sds[g][j] = pltpu.async_copy(
                    ones_v, acc_c.at[dstb[p].at[j]], ssem[p], add=True)
        for d in sds[n_grp - 2]:
            d.wait()
        for d in sds[n_grp - 1]:
            d.wait()
        plsc.subcore_barrier()

        for j in range(n_blk):
            pltpu.sync_copy(acc_c.at[pl.ds(r0 + j * CHUNK, CHUNK)], st_v)
            pltpu.sync_copy(st_v,
                            cnt_hbm.at[c, pl.ds(r0 + j * CHUNK, CHUNK)])

    return body(dst_g, jnp.zeros((CHUNK, 16), jnp.float32),
                jnp.ones((CHUNK, 16), jnp.float32))


def kernel(x, edge_index, W_l0, b_l0, W_r0, W_l1, b_l1, W_r1,
           W_l2, b_l2, W_r2):
    ei = edge_index.astype(jnp.int32)
    n_pad = EP - ei.shape[1]
    pad_i = jnp.arange(n_pad, dtype=jnp.int32)
    src_g = jnp.concatenate(
        [ei[0], pad_i % 4096]).reshape(NW, CPT, CHUNK)
    dst_g = jnp.concatenate(
        [ei[1], N_NODES + pad_i % (NP - N_NODES)]).reshape(NW, CPT, CHUNK)
    x_p = jnp.pad(x, ((0, NP - N_NODES), (0, 0)))

    Y0, R0 = _matmuls(x_p, W_l0, W_r0, b_l0)
    (C,) = _seg_cnt(dst_g)
    (P0,) = _seg_sum(Y0, src_g, dst_g)
    Y1, R1 = _combine_mm(P0, C, R0, W_l1, W_r1, b_l1)
    (P1,) = _seg_sum(Y1, src_g, dst_g)
    Wl2 = jnp.pad(W_l2, ((0, 0), (0, 8)))
    Wr2 = jnp.pad(W_r2, ((0, 0), (0, 8)))
    b2 = jnp.pad(b_l2, (0, 8))
    Y2, R2 = _combine_mm(P1, C, R1, Wl2, Wr2, b2)
    (P2,) = _seg_sum(Y2, src_g, dst_g)
    out = _combine(P2, C, R2, "logsoftmax")
    return out[:N_NODES, :40]


# speedup vs baseline: 1.0396x; 1.0396x over previous
import functools

import jax
import jax.numpy as jnp
from jax import lax
from jax.experimental import pallas as pl
from jax.experimental.pallas import tpu as pltpu
from jax.experimental.pallas import tpu_sc as plsc

N_NODES = 10000
NP = 10240
N_CORES = 2
N_SUBCORES = 16
NW = N_CORES * N_SUBCORES
CHUNK = 128
CPT = 80
GRP = 16
EP = NW * CPT * CHUNK
ROWS_PER_TILE = NP // N_SUBCORES


def _mm_body(h_ref, wl_ref, wr_ref, b_ref, y_ref, r_ref):
    h = h_ref[...]
    y_ref[...] = jnp.dot(h, wl_ref[...], preferred_element_type=jnp.float32)
    r_ref[...] = (jnp.dot(h, wr_ref[...], preferred_element_type=jnp.float32)
                  + b_ref[0:1, :])


def _matmuls(h, W_l, W_r, b):
    ci, co = W_l.shape
    blk = 512
    b8 = jnp.broadcast_to(b.reshape(1, co), (8, co))
    return pl.pallas_call(
        _mm_body,
        grid=(NP // blk,),
        in_specs=[
            pl.BlockSpec((blk, ci), lambda i: (i, 0)),
            pl.BlockSpec((ci, co), lambda i: (0, 0)),
            pl.BlockSpec((ci, co), lambda i: (0, 0)),
            pl.BlockSpec((8, co), lambda i: (0, 0)),
        ],
        out_specs=[
            pl.BlockSpec((blk, co), lambda i: (i, 0)),
            pl.BlockSpec((blk, co), lambda i: (i, 0)),
        ],
        out_shape=[
            jax.ShapeDtypeStruct((NP, co), jnp.float32),
            jax.ShapeDtypeStruct((NP, co), jnp.float32),
        ],
    )(h, W_l, W_r, b8)


def _cmm_body(p_ref, c_ref, r_ref, wl_ref, wr_ref, b_ref, y_ref, r2_ref):
    p = p_ref[0] + p_ref[1]
    cnt = c_ref[0, :, 0:1] + c_ref[1, :, 0:1]
    inv = 1.0 / jnp.maximum(cnt, 1.0)
    h = jnp.maximum(p * inv + r_ref[...], 0.0)
    y_ref[...] = jnp.dot(h, wl_ref[...], preferred_element_type=jnp.float32)
    r2_ref[...] = (jnp.dot(h, wr_ref[...], preferred_element_type=jnp.float32)
                   + b_ref[0:1, :])


def _combine_mm(P, C, R, W_l, W_r, b):
    ci = P.shape[2]
    co = W_l.shape[1]
    blk = 512
    b8 = jnp.broadcast_to(b.reshape(1, co), (8, co))
    return pl.pallas_call(
        _cmm_body,
        grid=(NP // blk,),
        in_specs=[
            pl.BlockSpec((2, blk, ci), lambda i: (0, i, 0)),
            pl.BlockSpec((2, blk, 16), lambda i: (0, i, 0)),
            pl.BlockSpec((blk, ci), lambda i: (i, 0)),
            pl.BlockSpec((ci, co), lambda i: (0, 0)),
            pl.BlockSpec((ci, co), lambda i: (0, 0)),
            pl.BlockSpec((8, co), lambda i: (0, 0)),
        ],
        out_specs=[
            pl.BlockSpec((blk, co), lambda i: (i, 0)),
            pl.BlockSpec((blk, co), lambda i: (i, 0)),
        ],
        out_shape=[
            jax.ShapeDtypeStruct((NP, co), jnp.float32),
            jax.ShapeDtypeStruct((NP, co), jnp.float32),
        ],
    )(P, C, R, W_l, W_r, b8)


def _combine_body(p_ref, c_ref, r_ref, o_ref, *, act):
    p = p_ref[0] + p_ref[1]
    cnt = c_ref[0, :, 0:1] + c_ref[1, :, 0:1]
    inv = 1.0 / jnp.maximum(cnt, 1.0)
    z = p * inv + r_ref[...]
    if act == "relu":
        o_ref[...] = jnp.maximum(z, 0.0)
    else:
        col = lax.broadcasted_iota(jnp.int32, z.shape, 1)
        valid = col < 40
        zm = jnp.where(valid, z, -jnp.inf)
        m = jnp.max(zm, axis=1, keepdims=True)
        e = jnp.where(valid, jnp.exp(z - m), 0.0)
        lse = jnp.log(jnp.sum(e, axis=1, keepdims=True)) + m
        o_ref[...] = z - lse


def _combine(P, C, R, act):
    co = P.shape[2]
    blk = 512
    return pl.pallas_call(
        functools.partial(_combine_body, act=act),
        grid=(NP // blk,),
        in_specs=[
            pl.BlockSpec((2, blk, co), lambda i: (0, i, 0)),
            pl.BlockSpec((2, blk, 16), lambda i: (0, i, 0)),
            pl.BlockSpec((blk, co), lambda i: (i, 0)),
        ],
        out_specs=pl.BlockSpec((blk, co), lambda i: (i, 0)),
        out_shape=jax.ShapeDtypeStruct((NP, co), jnp.float32),
    )(P, C, R)


def _seg_sum(y, src_g, dst_g):
    co = y.shape[1]
    mesh = plsc.VectorSubcoreMesh(
        core_axis_name="c", subcore_axis_name="s",
        num_cores=N_CORES, num_subcores=N_SUBCORES)

    out_type = [jax.ShapeDtypeStruct((N_CORES, NP, co), jnp.float32)]
    scratch = [
        pltpu.VMEM((GRP, CHUNK), jnp.int32),
        pltpu.VMEM((GRP, CHUNK), jnp.int32),
        pltpu.VMEM((CHUNK, co), jnp.float32),
        pltpu.VMEM((CHUNK, co), jnp.float32),
        pltpu.VMEM_SHARED((NP, co), jnp.float32),
        pltpu.SemaphoreType.DMA,
        pltpu.SemaphoreType.DMA,
        pltpu.SemaphoreType.DMA,
        pltpu.SemaphoreType.DMA,
    ]

    zeros = jnp.zeros((CHUNK, co), jnp.float32)
    ins = [y, src_g, dst_g, zeros]

    @functools.partial(
        pl.kernel, out_type=out_type, mesh=mesh, scratch_types=scratch,
        compiler_params=pltpu.CompilerParams(use_tc_tiling_on_sc=(co == 128)))
    def body(y_hbm, src_hbm, dst_hbm, z_hbm, out_hbm,
             src_v, dst_v, rows_a, rows_b, acc, gsa, gsb, ssa, ssb):
        c = lax.axis_index("c")
        s = lax.axis_index("s")
        wid = c * N_SUBCORES + s
        r0 = s * ROWS_PER_TILE
        n_blk = ROWS_PER_TILE // CHUNK
        rows = (rows_a, rows_b)
        gsem = (gsa, gsb)
        ssem = (ssa, ssb)
        pltpu.sync_copy(z_hbm, rows_a)
        zds = [pltpu.async_copy(rows_a, acc.at[pl.ds(r0 + j * CHUNK, CHUNK)],
                                gsa)
               for j in range(n_blk)]
        for d in zds:
            d.wait()
        plsc.subcore_barrier()

        def group(g, carry):
            pltpu.sync_copy(src_hbm.at[wid, pl.ds(g * GRP, GRP)], src_v)
            pltpu.sync_copy(dst_hbm.at[wid, pl.ds(g * GRP, GRP)], dst_v)
            gd = [None, None]
            sd = [None, None]
            gd[0] = pltpu.async_copy(y_hbm.at[src_v.at[0]], rows[0], gsem[0])
            for j in range(GRP):
                p = j % 2
                q = 1 - p
                if j + 1 < GRP:
                    if j >= 1:
                        sd[q].wait()
                    gd[q] = pltpu.async_copy(
                        y_hbm.at[src_v.at[j + 1]], rows[q], gsem[q])
                gd[p].wait()
                sd[p] = pltpu.async_copy(
                    rows[p], acc.at[dst_v.at[j]], ssem[p], add=True)
            sd[0].wait()
            sd[1].wait()
            return carry

        lax.fori_loop(0, CPT // GRP, group, 0)
        plsc.subcore_barrier()

        ids = [None] * n_blk
        ods = [None] * n_blk
        ids[0] = pltpu.async_copy(acc.at[pl.ds(r0, CHUNK)], rows_a, gsa)
        for j in range(n_blk):
            p = j % 2
            q = 1 - p
            if j + 1 < n_blk:
                if j >= 1:
                    ods[j - 1].wait()
                ids[j + 1] = pltpu.async_copy(
                    acc.at[pl.ds(r0 + (j + 1) * CHUNK, CHUNK)],
                    rows[q], gsem[q])
            ids[j].wait()
            ods[j] = pltpu.async_copy(
                rows[p], out_hbm.at[c, pl.ds(r0 + j * CHUNK, CHUNK)],
                ssem[p])
        ods[n_blk - 2].wait()
        ods[n_blk - 1].wait()

    return body(*ins)


def _seg_cnt(dst_g):
    mesh = plsc.VectorSubcoreMesh(
        core_axis_name="c", subcore_axis_name="s",
        num_cores=N_CORES, num_subcores=N_SUBCORES)

    out_type = [jax.ShapeDtypeStruct((N_CORES, NP, 16), jnp.float32)]
    scratch = [
        pltpu.VMEM((GRP, CHUNK), jnp.int32),
        pltpu.VMEM((GRP, CHUNK), jnp.int32),
        pltpu.VMEM((CHUNK, 16), jnp.float32),
        pltpu.VMEM((CHUNK, 16), jnp.float32),
        pltpu.VMEM_SHARED((NP, 16), jnp.float32),
        pltpu.SemaphoreType.DMA,
        pltpu.SemaphoreType.DMA,
        pltpu.SemaphoreType.DMA,
        pltpu.SemaphoreType.DMA,
    ]

    @functools.partial(
        pl.kernel, out_type=out_type, mesh=mesh, scratch_types=scratch,
        compiler_params=pltpu.CompilerParams(use_tc_tiling_on_sc=False))
    def body(dst_hbm, z_hbm, ones_hbm, cnt_hbm,
             dst_a, dst_b, ones_v, st_v, acc_c, isa, isb, ssa, ssb):
        c = lax.axis_index("c")
        s = lax.axis_index("s")
        wid = c * N_SUBCORES + s
        r0 = s * ROWS_PER_TILE
        n_blk = ROWS_PER_TILE // CHUNK
        n_grp = CPT // GRP
        dstb = (dst_a, dst_b)
        isem = (isa, isb)
        ssem = (ssa, ssb)
        pltpu.sync_copy(z_hbm, st_v)
        pltpu.sync_copy(ones_hbm, ones_v)
        zds = [pltpu.async_copy(st_v, acc_c.at[pl.ds(r0 + j * CHUNK, CHUNK)],
                                ssa)
               for j in range(n_blk)]
        for d in zds:
            d.wait()
        plsc.subcore_barrier()

        ids = [None] * n_grp
        sds = [[None] * GRP for _ in range(n_grp)]
        ids[0] = pltpu.async_copy(dst_hbm.at[wid, pl.ds(0, GRP)], dst_a,
                                  isem[0])
        for g in range(n_grp):
            p = g % 2
            if g + 1 < n_grp:
                if g >= 1:
                    for d in sds[g - 1]:
                        d.wait()
                ids[g + 1] = pltpu.async_copy(
                    dst_hbm.at[wid, pl.ds((g + 1) * GRP, GRP)],
                    dstb[1 - p], isem[1 - p])
            ids[g].wait()
            for j in range(GRP):
                sds[g][j] = pltpu.async_copy(
                    ones_v, acc_c.at[dstb[p].at[j]], ssem[p], add=True)
        for d in sds[n_grp - 2]:
            d.wait()
        for d in sds[n_grp - 1]:
            d.wait()
        plsc.subcore_barrier()

        for j in range(n_blk):
            pltpu.sync_copy(acc_c.at[pl.ds(r0 + j * CHUNK, CHUNK)], st_v)
            pltpu.sync_copy(st_v,
                            cnt_hbm.at[c, pl.ds(r0 + j * CHUNK, CHUNK)])

    return body(dst_g, jnp.zeros((CHUNK, 16), jnp.float32),
                jnp.ones((CHUNK, 16), jnp.float32))


def kernel(x, edge_index, W_l0, b_l0, W_r0, W_l1, b_l1, W_r1,
           W_l2, b_l2, W_r2):
    ei = edge_index.astype(jnp.int32)
    n_pad = EP - ei.shape[1]
    pad_i = jnp.arange(n_pad, dtype=jnp.int32)
    src_g = jnp.concatenate(
        [ei[0], pad_i % 4096]).reshape(NW, CPT, CHUNK)
    dst_g = jnp.concatenate(
        [ei[1], N_NODES + pad_i % (NP - N_NODES)]).reshape(NW, CPT, CHUNK)
    x_p = jnp.pad(x, ((0, NP - N_NODES), (0, 0)))

    Y0, R0 = _matmuls(x_p, W_l0, W_r0, b_l0)
    (C,) = _seg_cnt(dst_g)
    (P0,) = _seg_sum(Y0, src_g, dst_g)
    Y1, R1 = _combine_mm(P0, C, R0, W_l1, W_r1, b_l1)
    (P1,) = _seg_sum(Y1, src_g, dst_g)
    Wl2 = jnp.pad(W_l2, ((0, 0), (0, 8)))
    Wr2 = jnp.pad(W_r2, ((0, 0), (0, 8)))
    b2 = jnp.pad(b_l2, (0, 8))
    Y2, R2 = _combine_mm(P1, C, R1, Wl2, Wr2, b2)
    (P2,) = _seg_sum(Y2, src_g, dst_g)
    out = _combine(P2, C, R2, "logsoftmax")
    return out[:N_NODES, :40]


# speedup vs baseline: 1.0491x; 1.0092x over previous
import functools

import jax
import jax.numpy as jnp
from jax import lax
from jax.experimental import pallas as pl
from jax.experimental.pallas import tpu as pltpu
from jax.experimental.pallas import tpu_sc as plsc

N_NODES = 10000
NP = 10240
N_CORES = 2
N_SUBCORES = 16
NW = N_CORES * N_SUBCORES
CHUNK = 128
CPT = 80
GRP = 16
EP = NW * CPT * CHUNK
ROWS_PER_TILE = NP // N_SUBCORES


def _mm_body(h_ref, wl_ref, wr_ref, b_ref, y_ref, r_ref):
    h = h_ref[...]
    y_ref[...] = jnp.dot(h, wl_ref[...], preferred_element_type=jnp.float32)
    r_ref[...] = (jnp.dot(h, wr_ref[...], preferred_element_type=jnp.float32)
                  + b_ref[0:1, :])


def _matmuls(h, W_l, W_r, b):
    ci, co = W_l.shape
    blk = 512
    b8 = jnp.broadcast_to(b.reshape(1, co), (8, co))
    return pl.pallas_call(
        _mm_body,
        grid=(NP // blk,),
        in_specs=[
            pl.BlockSpec((blk, ci), lambda i: (i, 0)),
            pl.BlockSpec((ci, co), lambda i: (0, 0)),
            pl.BlockSpec((ci, co), lambda i: (0, 0)),
            pl.BlockSpec((8, co), lambda i: (0, 0)),
        ],
        out_specs=[
            pl.BlockSpec((blk, co), lambda i: (i, 0)),
            pl.BlockSpec((blk, co), lambda i: (i, 0)),
        ],
        out_shape=[
            jax.ShapeDtypeStruct((NP, co), jnp.float32),
            jax.ShapeDtypeStruct((NP, co), jnp.float32),
        ],
    )(h, W_l, W_r, b8)


def _cmm_body(p_ref, c_ref, r_ref, wl_ref, wr_ref, b_ref, y_ref, r2_ref):
    p = p_ref[0] + p_ref[1]
    cnt = c_ref[0, :, 0:1] + c_ref[1, :, 0:1]
    inv = 1.0 / jnp.maximum(cnt, 1.0)
    h = jnp.maximum(p * inv + r_ref[...], 0.0)
    y_ref[...] = jnp.dot(h, wl_ref[...], preferred_element_type=jnp.float32)
    r2_ref[...] = (jnp.dot(h, wr_ref[...], preferred_element_type=jnp.float32)
                   + b_ref[0:1, :])


def _combine_mm(P, C, R, W_l, W_r, b):
    ci = P.shape[2]
    co = W_l.shape[1]
    blk = 512
    b8 = jnp.broadcast_to(b.reshape(1, co), (8, co))
    return pl.pallas_call(
        _cmm_body,
        grid=(NP // blk,),
        in_specs=[
            pl.BlockSpec((2, blk, ci), lambda i: (0, i, 0)),
            pl.BlockSpec((2, blk, 16), lambda i: (0, i, 0)),
            pl.BlockSpec((blk, ci), lambda i: (i, 0)),
            pl.BlockSpec((ci, co), lambda i: (0, 0)),
            pl.BlockSpec((ci, co), lambda i: (0, 0)),
            pl.BlockSpec((8, co), lambda i: (0, 0)),
        ],
        out_specs=[
            pl.BlockSpec((blk, co), lambda i: (i, 0)),
            pl.BlockSpec((blk, co), lambda i: (i, 0)),
        ],
        out_shape=[
            jax.ShapeDtypeStruct((NP, co), jnp.float32),
            jax.ShapeDtypeStruct((NP, co), jnp.float32),
        ],
    )(P, C, R, W_l, W_r, b8)


def _combine_body(p_ref, c_ref, r_ref, o_ref):
    p = p_ref[0] + p_ref[1]
    cnt = c_ref[0, :, 0:1] + c_ref[1, :, 0:1]
    inv = 1.0 / jnp.maximum(cnt, 1.0)
    z = p * inv + r_ref[...]
    col = lax.broadcasted_iota(jnp.int32, z.shape, 1)
    valid = col < 40
    zm = jnp.where(valid, z, -jnp.inf)
    m = jnp.max(zm, axis=1, keepdims=True)
    e = jnp.where(valid, jnp.exp(z - m), 0.0)
    lse = jnp.log(jnp.sum(e, axis=1, keepdims=True)) + m
    o_ref[...] = (z - lse)[:, :40]


def _combine(P, C, R):
    co = P.shape[2]
    blk = 400
    return pl.pallas_call(
        _combine_body,
        grid=(N_NODES // blk,),
        in_specs=[
            pl.BlockSpec((2, blk, co), lambda i: (0, i, 0)),
            pl.BlockSpec((2, blk, 16), lambda i: (0, i, 0)),
            pl.BlockSpec((blk, co), lambda i: (i, 0)),
        ],
        out_specs=pl.BlockSpec((blk, 40), lambda i: (i, 0)),
        out_shape=jax.ShapeDtypeStruct((N_NODES, 40), jnp.float32),
    )(P, C, R)


def _seg_sum(y, e_g):
    co = y.shape[1]
    mesh = plsc.VectorSubcoreMesh(
        core_axis_name="c", subcore_axis_name="s",
        num_cores=N_CORES, num_subcores=N_SUBCORES)

    out_type = [jax.ShapeDtypeStruct((N_CORES, NP, co), jnp.float32)]
    scratch = [
        pltpu.VMEM((GRP, CHUNK), jnp.int32),
        pltpu.VMEM((GRP, CHUNK), jnp.int32),
        pltpu.VMEM((CHUNK, co), jnp.float32),
        pltpu.VMEM((CHUNK, co), jnp.float32),
        pltpu.VMEM_SHARED((NP, co), jnp.float32),
        pltpu.SemaphoreType.DMA,
        pltpu.SemaphoreType.DMA,
        pltpu.SemaphoreType.DMA,
        pltpu.SemaphoreType.DMA,
    ]

    zeros = jnp.zeros((CHUNK, co), jnp.float32)
    ins = [y, e_g, zeros]

    @functools.partial(
        pl.kernel, out_type=out_type, mesh=mesh, scratch_types=scratch,
        compiler_params=pltpu.CompilerParams(use_tc_tiling_on_sc=(co == 128)))
    def body(y_hbm, e_hbm, z_hbm, out_hbm,
             src_v, dst_v, rows_a, rows_b, acc, gsa, gsb, ssa, ssb):
        c = lax.axis_index("c")
        s = lax.axis_index("s")
        wid = c * N_SUBCORES + s
        r0 = s * ROWS_PER_TILE
        n_blk = ROWS_PER_TILE // CHUNK
        rows = (rows_a, rows_b)
        gsem = (gsa, gsb)
        ssem = (ssa, ssb)
        pltpu.sync_copy(z_hbm, rows_a)
        zds = [pltpu.async_copy(rows_a, acc.at[pl.ds(r0 + j * CHUNK, CHUNK)],
                                gsa)
               for j in range(n_blk)]
        for d in zds:
            d.wait()
        plsc.subcore_barrier()

        def group(g, carry):
            pltpu.sync_copy(e_hbm.at[0, wid, pl.ds(g * GRP, GRP)], src_v)
            pltpu.sync_copy(e_hbm.at[1, wid, pl.ds(g * GRP, GRP)], dst_v)
            gd = [None, None]
            sd = [None, None]
            gd[0] = pltpu.async_copy(y_hbm.at[src_v.at[0]], rows[0], gsem[0])
            for j in range(GRP):
                p = j % 2
                q = 1 - p
                if j + 1 < GRP:
                    if j >= 1:
                        sd[q].wait()
                    gd[q] = pltpu.async_copy(
                        y_hbm.at[src_v.at[j + 1]], rows[q], gsem[q])
                gd[p].wait()
                sd[p] = pltpu.async_copy(
                    rows[p], acc.at[dst_v.at[j]], ssem[p], add=True)
            sd[0].wait()
            sd[1].wait()
            return carry

        lax.fori_loop(0, CPT // GRP, group, 0)
        plsc.subcore_barrier()

        ids = [None] * n_blk
        ods = [None] * n_blk
        ids[0] = pltpu.async_copy(acc.at[pl.ds(r0, CHUNK)], rows_a, gsa)
        for j in range(n_blk):
            p = j % 2
            q = 1 - p
            if j + 1 < n_blk:
                if j >= 1:
                    ods[j - 1].wait()
                ids[j + 1] = pltpu.async_copy(
                    acc.at[pl.ds(r0 + (j + 1) * CHUNK, CHUNK)],
                    rows[q], gsem[q])
            ids[j].wait()
            ods[j] = pltpu.async_copy(
                rows[p], out_hbm.at[c, pl.ds(r0 + j * CHUNK, CHUNK)],
                ssem[p])
        ods[n_blk - 2].wait()
        ods[n_blk - 1].wait()

    return body(*ins)


def _seg_cnt(e_g):
    mesh = plsc.VectorSubcoreMesh(
        core_axis_name="c", subcore_axis_name="s",
        num_cores=N_CORES, num_subcores=N_SUBCORES)

    out_type = [jax.ShapeDtypeStruct((N_CORES, NP, 16), jnp.float32)]
    scratch = [
        pltpu.VMEM((GRP, CHUNK), jnp.int32),
        pltpu.VMEM((GRP, CHUNK), jnp.int32),
        pltpu.VMEM((CHUNK, 16), jnp.float32),
        pltpu.VMEM((CHUNK, 16), jnp.float32),
        pltpu.VMEM_SHARED((NP, 16), jnp.float32),
        pltpu.SemaphoreType.DMA,
        pltpu.SemaphoreType.DMA,
        pltpu.SemaphoreType.DMA,
        pltpu.SemaphoreType.DMA,
    ]

    @functools.partial(
        pl.kernel, out_type=out_type, mesh=mesh, scratch_types=scratch,
        compiler_params=pltpu.CompilerParams(use_tc_tiling_on_sc=False))
    def body(e_hbm, z_hbm, ones_hbm, cnt_hbm,
             dst_a, dst_b, ones_v, st_v, acc_c, isa, isb, ssa, ssb):
        c = lax.axis_index("c")
        s = lax.axis_index("s")
        wid = c * N_SUBCORES + s
        r0 = s * ROWS_PER_TILE
        n_blk = ROWS_PER_TILE // CHUNK
        n_grp = CPT // GRP
        dstb = (dst_a, dst_b)
        isem = (isa, isb)
        ssem = (ssa, ssb)
        pltpu.sync_copy(z_hbm, st_v)
        pltpu.sync_copy(ones_hbm, ones_v)
        zds = [pltpu.async_copy(st_v, acc_c.at[pl.ds(r0 + j * CHUNK, CHUNK)],
                                ssa)
               for j in range(n_blk)]
        for d in zds:
            d.wait()
        plsc.subcore_barrier()

        ids = [None] * n_grp
        sds = [[None] * GRP for _ in range(n_grp)]
        ids[0] = pltpu.async_copy(e_hbm.at[1, wid, pl.ds(0, GRP)], dst_a,
                                  isem[0])
        for g in range(n_grp):
            p = g % 2
            if g + 1 < n_grp:
                if g >= 1:
                    for d in sds[g - 1]:
                        d.wait()
                ids[g + 1] = pltpu.async_copy(
                    e_hbm.at[1, wid, pl.ds((g + 1) * GRP, GRP)],
                    dstb[1 - p], isem[1 - p])
            ids[g].wait()
            for j in range(GRP):
                sds[g][j] = pltpu.async_copy(
                    ones_v, acc_c.at[dstb[p].at[j]], ssem[p], add=True)
        for d in sds[n_grp - 2]:
            d.wait()
        for d in sds[n_grp - 1]:
            d.wait()
        plsc.subcore_barrier()

        for j in range(n_blk):
            pltpu.sync_copy(acc_c.at[pl.ds(r0 + j * CHUNK, CHUNK)], st_v)
            pltpu.sync_copy(st_v,
                            cnt_hbm.at[c, pl.ds(r0 + j * CHUNK, CHUNK)])

    return body(e_g, jnp.zeros((CHUNK, 16), jnp.float32),
                jnp.ones((CHUNK, 16), jnp.float32))


def kernel(x, edge_index, W_l0, b_l0, W_r0, W_l1, b_l1, W_r1,
           W_l2, b_l2, W_r2):
    ei = edge_index.astype(jnp.int32)
    n_pad = EP - ei.shape[1]
    pad_i = jnp.arange(n_pad, dtype=jnp.int32)
    pad_block = jnp.stack([pad_i % 4096, N_NODES + pad_i % (NP - N_NODES)])
    e_g = jnp.concatenate([ei, pad_block], axis=1).reshape(2, NW, CPT, CHUNK)
    x_p = jnp.pad(x, ((0, NP - N_NODES), (0, 0)))

    Y0, R0 = _matmuls(x_p, W_l0, W_r0, b_l0)
    (C,) = _seg_cnt(e_g)
    (P0,) = _seg_sum(Y0, e_g)
    Y1, R1 = _combine_mm(P0, C, R0, W_l1, W_r1, b_l1)
    (P1,) = _seg_sum(Y1, e_g)
    Wl2 = jnp.pad(W_l2, ((0, 0), (0, 8)))
    Wr2 = jnp.pad(W_r2, ((0, 0), (0, 8)))
    b2 = jnp.pad(b_l2, (0, 8))
    Y2, R2 = _combine_mm(P1, C, R1, Wl2, Wr2, b2)
    (P2,) = _seg_sum(Y2, e_g)
    return _combine(P2, C, R2)


# speedup vs baseline: 1.0855x; 1.0347x over previous
import functools

import jax
import jax.numpy as jnp
from jax import lax
from jax.experimental import pallas as pl
from jax.experimental.pallas import tpu as pltpu
from jax.experimental.pallas import tpu_sc as plsc

N_NODES = 10000
NP = 10240
N_CORES = 2
N_SUBCORES = 16
NW = N_CORES * N_SUBCORES
CHUNK = 128
CPT = 80
GRP = 16
EP = NW * CPT * CHUNK
ROWS_PER_TILE = NP // N_SUBCORES


def _mm_body(h_ref, w_ref, b_ref, y_ref, r_ref):
    h = h_ref[...]
    co = y_ref.shape[1]
    yr = jnp.dot(h, w_ref[...], preferred_element_type=jnp.float32)
    y_ref[...] = yr[:, :co]
    r_ref[...] = yr[:, co:] + b_ref[0:1, :]


def _matmuls(h, W_l, W_r, b):
    ci, co = W_l.shape
    blk = 1024
    w = jnp.concatenate([W_l, W_r], axis=1)
    b8 = jnp.broadcast_to(b.reshape(1, co), (8, co))
    return pl.pallas_call(
        _mm_body,
        grid=(NP // blk,),
        in_specs=[
            pl.BlockSpec((blk, ci), lambda i: (i, 0)),
            pl.BlockSpec((ci, 2 * co), lambda i: (0, 0)),
            pl.BlockSpec((8, co), lambda i: (0, 0)),
        ],
        out_specs=[
            pl.BlockSpec((blk, co), lambda i: (i, 0)),
            pl.BlockSpec((blk, co), lambda i: (i, 0)),
        ],
        out_shape=[
            jax.ShapeDtypeStruct((NP, co), jnp.float32),
            jax.ShapeDtypeStruct((NP, co), jnp.float32),
        ],
    )(h, w, b8)


def _cmm_body(p_ref, c_ref, r_ref, w_ref, b_ref, y_ref, r2_ref):
    p = p_ref[0] + p_ref[1]
    cnt = c_ref[0, :, 0:1] + c_ref[1, :, 0:1]
    inv = 1.0 / jnp.maximum(cnt, 1.0)
    h = jnp.maximum(p * inv + r_ref[...], 0.0)
    co = y_ref.shape[1]
    yr = jnp.dot(h, w_ref[...], preferred_element_type=jnp.float32)
    y_ref[...] = yr[:, :co]
    r2_ref[...] = yr[:, co:] + b_ref[0:1, :]


def _combine_mm(P, C, R, W_l, W_r, b):
    ci = P.shape[2]
    co = W_l.shape[1]
    blk = 1024
    w = jnp.concatenate([W_l, W_r], axis=1)
    b8 = jnp.broadcast_to(b.reshape(1, co), (8, co))
    return pl.pallas_call(
        _cmm_body,
        grid=(NP // blk,),
        in_specs=[
            pl.BlockSpec((2, blk, ci), lambda i: (0, i, 0)),
            pl.BlockSpec((2, blk, 16), lambda i: (0, i, 0)),
            pl.BlockSpec((blk, ci), lambda i: (i, 0)),
            pl.BlockSpec((ci, 2 * co), lambda i: (0, 0)),
            pl.BlockSpec((8, co), lambda i: (0, 0)),
        ],
        out_specs=[
            pl.BlockSpec((blk, co), lambda i: (i, 0)),
            pl.BlockSpec((blk, co), lambda i: (i, 0)),
        ],
        out_shape=[
            jax.ShapeDtypeStruct((NP, co), jnp.float32),
            jax.ShapeDtypeStruct((NP, co), jnp.float32),
        ],
    )(P, C, R, w, b8)


def _combine_body(p_ref, c_ref, r_ref, o_ref):
    p = p_ref[0] + p_ref[1]
    cnt = c_ref[0, :, 0:1] + c_ref[1, :, 0:1]
    inv = 1.0 / jnp.maximum(cnt, 1.0)
    z = p * inv + r_ref[...]
    col = lax.broadcasted_iota(jnp.int32, z.shape, 1)
    valid = col < 40
    zm = jnp.where(valid, z, -jnp.inf)
    m = jnp.max(zm, axis=1, keepdims=True)
    e = jnp.where(valid, jnp.exp(z - m), 0.0)
    lse = jnp.log(jnp.sum(e, axis=1, keepdims=True)) + m
    o_ref[...] = (z - lse)[:, :40]


def _combine(P, C, R):
    co = P.shape[2]
    blk = 400
    return pl.pallas_call(
        _combine_body,
        grid=(N_NODES // blk,),
        in_specs=[
            pl.BlockSpec((2, blk, co), lambda i: (0, i, 0)),
            pl.BlockSpec((2, blk, 16), lambda i: (0, i, 0)),
            pl.BlockSpec((blk, co), lambda i: (i, 0)),
        ],
        out_specs=pl.BlockSpec((blk, 40), lambda i: (i, 0)),
        out_shape=jax.ShapeDtypeStruct((N_NODES, 40), jnp.float32),
    )(P, C, R)


def _seg_sum(y, e_g):
    co = y.shape[1]
    mesh = plsc.VectorSubcoreMesh(
        core_axis_name="c", subcore_axis_name="s",
        num_cores=N_CORES, num_subcores=N_SUBCORES)

    out_type = [jax.ShapeDtypeStruct((N_CORES, NP, co), jnp.float32)]
    scratch = [
        pltpu.VMEM((GRP, CHUNK), jnp.int32),
        pltpu.VMEM((GRP, CHUNK), jnp.int32),
        pltpu.VMEM((CHUNK, co), jnp.float32),
        pltpu.VMEM((CHUNK, co), jnp.float32),
        pltpu.VMEM_SHARED((NP, co), jnp.float32),
        pltpu.SemaphoreType.DMA,
        pltpu.SemaphoreType.DMA,
        pltpu.SemaphoreType.DMA,
        pltpu.SemaphoreType.DMA,
    ]

    zeros = jnp.zeros((CHUNK, co), jnp.float32)
    ins = [y, e_g, zeros]

    @functools.partial(
        pl.kernel, out_type=out_type, mesh=mesh, scratch_types=scratch,
        compiler_params=pltpu.CompilerParams(use_tc_tiling_on_sc=(co == 128)))
    def body(y_hbm, e_hbm, z_hbm, out_hbm,
             src_v, dst_v, rows_a, rows_b, acc, gsa, gsb, ssa, ssb):
        c = lax.axis_index("c")
        s = lax.axis_index("s")
        wid = c * N_SUBCORES + s
        r0 = s * ROWS_PER_TILE
        n_blk = ROWS_PER_TILE // CHUNK
        rows = (rows_a, rows_b)
        gsem = (gsa, gsb)
        ssem = (ssa, ssb)
        pltpu.sync_copy(z_hbm, rows_a)
        zds = [pltpu.async_copy(rows_a, acc.at[pl.ds(r0 + j * CHUNK, CHUNK)],
                                gsa)
               for j in range(n_blk)]
        for d in zds:
            d.wait()
        plsc.subcore_barrier()

        def group(g, carry):
            pltpu.sync_copy(e_hbm.at[0, wid, pl.ds(g * GRP, GRP)], src_v)
            pltpu.sync_copy(e_hbm.at[1, wid, pl.ds(g * GRP, GRP)], dst_v)
            gd = [None, None]
            sd = [None, None]
            gd[0] = pltpu.async_copy(y_hbm.at[src_v.at[0]], rows[0], gsem[0])
            for j in range(GRP):
                p = j % 2
                q = 1 - p
                if j + 1 < GRP:
                    if j >= 1:
                        sd[q].wait()
                    gd[q] = pltpu.async_copy(
                        y_hbm.at[src_v.at[j + 1]], rows[q], gsem[q])
                gd[p].wait()
                sd[p] = pltpu.async_copy(
                    rows[p], acc.at[dst_v.at[j]], ssem[p], add=True)
            sd[0].wait()
            sd[1].wait()
            return carry

        lax.fori_loop(0, CPT // GRP, group, 0)
        plsc.subcore_barrier()

        ids = [None] * n_blk
        ods = [None] * n_blk
        ids[0] = pltpu.async_copy(acc.at[pl.ds(r0, CHUNK)], rows_a, gsa)
        for j in range(n_blk):
            p = j % 2
            q = 1 - p
            if j + 1 < n_blk:
                if j >= 1:
                    ods[j - 1].wait()
                ids[j + 1] = pltpu.async_copy(
                    acc.at[pl.ds(r0 + (j + 1) * CHUNK, CHUNK)],
                    rows[q], gsem[q])
            ids[j].wait()
            ods[j] = pltpu.async_copy(
                rows[p], out_hbm.at[c, pl.ds(r0 + j * CHUNK, CHUNK)],
                ssem[p])
        ods[n_blk - 2].wait()
        ods[n_blk - 1].wait()

    return body(*ins)


def _seg_cnt(e_g):
    mesh = plsc.VectorSubcoreMesh(
        core_axis_name="c", subcore_axis_name="s",
        num_cores=N_CORES, num_subcores=N_SUBCORES)

    out_type = [jax.ShapeDtypeStruct((N_CORES, NP, 16), jnp.float32)]
    scratch = [
        pltpu.VMEM((GRP, CHUNK), jnp.int32),
        pltpu.VMEM((GRP, CHUNK), jnp.int32),
        pltpu.VMEM((CHUNK, 16), jnp.float32),
        pltpu.VMEM((CHUNK, 16), jnp.float32),
        pltpu.VMEM_SHARED((NP, 16), jnp.float32),
        pltpu.SemaphoreType.DMA,
        pltpu.SemaphoreType.DMA,
        pltpu.SemaphoreType.DMA,
        pltpu.SemaphoreType.DMA,
    ]

    @functools.partial(
        pl.kernel, out_type=out_type, mesh=mesh, scratch_types=scratch,
        compiler_params=pltpu.CompilerParams(use_tc_tiling_on_sc=False))
    def body(e_hbm, z_hbm, ones_hbm, cnt_hbm,
             dst_a, dst_b, ones_v, st_v, acc_c, isa, isb, ssa, ssb):
        c = lax.axis_index("c")
        s = lax.axis_index("s")
        wid = c * N_SUBCORES + s
        r0 = s * ROWS_PER_TILE
        n_blk = ROWS_PER_TILE // CHUNK
        n_grp = CPT // GRP
        dstb = (dst_a, dst_b)
        isem = (isa, isb)
        ssem = (ssa, ssb)
        pltpu.sync_copy(z_hbm, st_v)
        pltpu.sync_copy(ones_hbm, ones_v)
        zds = [pltpu.async_copy(st_v, acc_c.at[pl.ds(r0 + j * CHUNK, CHUNK)],
                                ssa)
               for j in range(n_blk)]
        for d in zds:
            d.wait()
        plsc.subcore_barrier()

        ids = [None] * n_grp
        sds = [[None] * GRP for _ in range(n_grp)]
        ids[0] = pltpu.async_copy(e_hbm.at[1, wid, pl.ds(0, GRP)], dst_a,
                                  isem[0])
        for g in range(n_grp):
            p = g % 2
            if g + 1 < n_grp:
                if g >= 1:
                    for d in sds[g - 1]:
                        d.wait()
                ids[g + 1] = pltpu.async_copy(
                    e_hbm.at[1, wid, pl.ds((g + 1) * GRP, GRP)],
                    dstb[1 - p], isem[1 - p])
            ids[g].wait()
            for j in range(GRP):
                sds[g][j] = pltpu.async_copy(
                    ones_v, acc_c.at[dstb[p].at[j]], ssem[p], add=True)
        for d in sds[n_grp - 2]:
            d.wait()
        for d in sds[n_grp - 1]:
            d.wait()
        plsc.subcore_barrier()

        for j in range(n_blk):
            pltpu.sync_copy(acc_c.at[pl.ds(r0 + j * CHUNK, CHUNK)], st_v)
            pltpu.sync_copy(st_v,
                            cnt_hbm.at[c, pl.ds(r0 + j * CHUNK, CHUNK)])

    return body(e_g, jnp.zeros((CHUNK, 16), jnp.float32),
                jnp.ones((CHUNK, 16), jnp.float32))


def kernel(x, edge_index, W_l0, b_l0, W_r0, W_l1, b_l1, W_r1,
           W_l2, b_l2, W_r2):
    ei = edge_index.astype(jnp.int32)
    n_pad = EP - ei.shape[1]
    pad_i = jnp.arange(n_pad, dtype=jnp.int32)
    pad_block = jnp.stack([pad_i % 4096, N_NODES + pad_i % (NP - N_NODES)])
    e_g = jnp.concatenate([ei, pad_block], axis=1).reshape(2, NW, CPT, CHUNK)
    x_p = jnp.pad(x, ((0, NP - N_NODES), (0, 0)))

    Y0, R0 = _matmuls(x_p, W_l0, W_r0, b_l0)
    (C,) = _seg_cnt(e_g)
    (P0,) = _seg_sum(Y0, e_g)
    Y1, R1 = _combine_mm(P0, C, R0, W_l1, W_r1, b_l1)
    (P1,) = _seg_sum(Y1, e_g)
    Wl2 = jnp.pad(W_l2, ((0, 0), (0, 8)))
    Wr2 = jnp.pad(W_r2, ((0, 0), (0, 8)))
    b2 = jnp.pad(b_l2, (0, 8))
    Y2, R2 = _combine_mm(P1, C, R1, Wl2, Wr2, b2)
    (P2,) = _seg_sum(Y2, e_g)
    return _combine(P2, C, R2)
